# Initial kernel scaffold; baseline (speedup 1.0000x reference)
import functools

import jax
import jax.numpy as jnp
from jax import lax
from jax.experimental import pallas as pl
from jax.experimental.pallas import tpu as pltpu
from jax.experimental.pallas import tpu_sc as plsc

B = 1024
NP_ = 200
D = 128
LABELS = 1000
T = B * NP_
NEG_INF = -2.0 * 10**10


def _transform2_body(tbl_ref, wa_ref, wb_ref, oa_ref, ob_ref):
    blk = tbl_ref[...]
    oa_ref[...] = jnp.dot(blk, wa_ref[...], preferred_element_type=jnp.float32)
    ob_ref[...] = jnp.dot(blk, wb_ref[...], preferred_element_type=jnp.float32)


def _transform1_body(tbl_ref, wa_ref, oa_ref):
    oa_ref[...] = jnp.dot(tbl_ref[...], wa_ref[...], preferred_element_type=jnp.float32)


def _transform_values(table, w_a, w_b, block_rows=2500):
    v = table.shape[0]
    grid = v // block_rows
    return pl.pallas_call(
        _transform2_body,
        grid=(grid,),
        in_specs=[
            pl.BlockSpec((block_rows, D), lambda i: (i, 0)),
            pl.BlockSpec((D, D), lambda i: (0, 0)),
            pl.BlockSpec((D, D), lambda i: (0, 0)),
        ],
        out_specs=[
            pl.BlockSpec((block_rows, D), lambda i: (i, 0)),
            pl.BlockSpec((block_rows, D), lambda i: (i, 0)),
        ],
        out_shape=[
            jax.ShapeDtypeStruct((v, D), jnp.float32),
            jax.ShapeDtypeStruct((v, D), jnp.float32),
        ],
    )(table, w_a, w_b)


def _transform_paths(table, w_a, block_rows=2500):
    v = table.shape[0]
    grid = v // block_rows
    return pl.pallas_call(
        _transform1_body,
        grid=(grid,),
        in_specs=[
            pl.BlockSpec((block_rows, D), lambda i: (i, 0)),
            pl.BlockSpec((D, D), lambda i: (0, 0)),
        ],
        out_specs=pl.BlockSpec((block_rows, D), lambda i: (i, 0)),
        out_shape=jax.ShapeDtypeStruct((v, D), jnp.float32),
    )(table, w_a)


_NC = 2
_NS = 16
_NW = _NC * _NS
_CHUNK = 128
_TPW = T // _NW
_NCHUNK = _TPW // _CHUNK


def _gather_add_sc(starts_f, paths_f, ends_f, tvs, tp, tve):
    mesh = plsc.VectorSubcoreMesh(core_axis_name="c", subcore_axis_name="s")

    @functools.partial(
        pl.kernel,
        mesh=mesh,
        out_type=jax.ShapeDtypeStruct((T, D), jnp.float32),
        scratch_types=[
            pltpu.VMEM((_CHUNK,), jnp.int32),
            pltpu.VMEM((_CHUNK,), jnp.int32),
            pltpu.VMEM((_CHUNK,), jnp.int32),
            pltpu.VMEM((_CHUNK, D), jnp.float32),
            pltpu.VMEM((_CHUNK, D), jnp.float32),
            pltpu.VMEM((_CHUNK, D), jnp.float32),
            pltpu.SemaphoreType.DMA,
            pltpu.SemaphoreType.DMA,
            pltpu.SemaphoreType.DMA,
        ],
    )
    def k(starts_hbm, paths_hbm, ends_hbm, tvs_hbm, tp_hbm, tve_hbm, out_hbm,
          idx_s, idx_p, idx_e, rs, rp, re_, sem1, sem2, sem3):
        wid = lax.axis_index("s") * _NC + lax.axis_index("c")
        wbase = wid * _TPW

        def chunk_body(ci, carry):
            base = wbase + ci * _CHUNK
            pltpu.sync_copy(starts_hbm.at[pl.ds(base, _CHUNK)], idx_s)
            pltpu.sync_copy(paths_hbm.at[pl.ds(base, _CHUNK)], idx_p)
            pltpu.sync_copy(ends_hbm.at[pl.ds(base, _CHUNK)], idx_e)
            c1 = pltpu.async_copy(tvs_hbm.at[idx_s], rs, sem1)
            c2 = pltpu.async_copy(tp_hbm.at[idx_p], rp, sem2)
            c3 = pltpu.async_copy(tve_hbm.at[idx_e], re_, sem3)
            c1.wait()
            c2.wait()
            c3.wait()

            def tok_body(t, acc):
                for j in range(D // 16):
                    sl = pl.ds(j * 16, 16)
                    rs[t, sl] = rs[t, sl] + rp[t, sl] + re_[t, sl]
                return acc

            lax.fori_loop(0, _CHUNK, tok_body, 0)
            pltpu.sync_copy(rs, out_hbm.at[pl.ds(base, _CHUNK)])
            return carry

        lax.fori_loop(0, _NCHUNK, chunk_body, 0)

    return k(starts_f, paths_f, ends_f, tvs, tp, tve)


_BB = 8


def _attn_body(pre_ref, starts_ref, a_ref, wout_ref, code_ref, out_ref):
    h = jnp.tanh(pre_ref[...])
    mask = (starts_ref[...] > 1).astype(jnp.float32)
    scores = jnp.sum(h * a_ref[...][None, :, :], axis=2)
    s = scores * mask + (1.0 - mask) * NEG_INF
    mx = jnp.max(s, axis=1, keepdims=True)
    e = jnp.exp(s - mx)
    w = e / jnp.sum(e, axis=1, keepdims=True)
    code = jnp.sum(h * w[:, :, None], axis=1)
    code_ref[...] = code
    out_ref[...] = jnp.dot(code, wout_ref[...], preferred_element_type=jnp.float32)


def _attention(pre, starts, a, w_out):
    grid = B // _BB
    return pl.pallas_call(
        _attn_body,
        grid=(grid,),
        in_specs=[
            pl.BlockSpec((_BB, NP_, D), lambda i: (i, 0, 0)),
            pl.BlockSpec((_BB, NP_), lambda i: (i, 0)),
            pl.BlockSpec((1, D), lambda i: (0, 0)),
            pl.BlockSpec((D, LABELS), lambda i: (0, 0)),
        ],
        out_specs=[
            pl.BlockSpec((_BB, D), lambda i: (i, 0)),
            pl.BlockSpec((_BB, LABELS), lambda i: (i, 0)),
        ],
        out_shape=[
            jax.ShapeDtypeStruct((B, D), jnp.float32),
            jax.ShapeDtypeStruct((B, LABELS), jnp.float32),
        ],
    )(pre, starts, a, w_out)


def kernel(starts, paths, ends, values_table, paths_table, W, a, W_out):
    w_s = W[0:D]
    w_p = W[D:2 * D]
    w_e = W[2 * D:3 * D]
    tvs, tve = _transform_values(values_table, w_s, w_e)
    tp = _transform_paths(paths_table, w_p)
    pre_flat = _gather_add_sc(
        starts.reshape(-1), paths.reshape(-1), ends.reshape(-1), tvs, tp, tve)
    pre = pre_flat.reshape(B, NP_, D)
    code, out = _attention(pre, starts, a, W_out)
    return (code, out)


# speedup vs baseline: 3.4760x; 3.4760x over previous
import functools

import jax
import jax.numpy as jnp
from jax import lax
from jax.experimental import pallas as pl
from jax.experimental.pallas import tpu as pltpu
from jax.experimental.pallas import tpu_sc as plsc

B = 1024
NP_ = 200
D = 128
LABELS = 1000
T = B * NP_
NEG_INF = -2.0 * 10**10


def _transform2_body(tbl_ref, wa_ref, wb_ref, oa_ref, ob_ref):
    blk = tbl_ref[...]
    oa_ref[...] = jnp.dot(blk, wa_ref[...], preferred_element_type=jnp.float32)
    ob_ref[...] = jnp.dot(blk, wb_ref[...], preferred_element_type=jnp.float32)


def _transform1_body(tbl_ref, wa_ref, oa_ref):
    oa_ref[...] = jnp.dot(tbl_ref[...], wa_ref[...], preferred_element_type=jnp.float32)


def _transform_values(table, w_a, w_b, block_rows=2000):
    v = table.shape[0]
    grid = v // block_rows
    return pl.pallas_call(
        _transform2_body,
        grid=(grid,),
        in_specs=[
            pl.BlockSpec((block_rows, D), lambda i: (i, 0)),
            pl.BlockSpec((D, D), lambda i: (0, 0)),
            pl.BlockSpec((D, D), lambda i: (0, 0)),
        ],
        out_specs=[
            pl.BlockSpec((block_rows, D), lambda i: (i, 0)),
            pl.BlockSpec((block_rows, D), lambda i: (i, 0)),
        ],
        out_shape=[
            jax.ShapeDtypeStruct((v, D), jnp.float32),
            jax.ShapeDtypeStruct((v, D), jnp.float32),
        ],
    )(table, w_a, w_b)


def _transform_paths(table, w_a, block_rows=2000):
    v = table.shape[0]
    grid = v // block_rows
    return pl.pallas_call(
        _transform1_body,
        grid=(grid,),
        in_specs=[
            pl.BlockSpec((block_rows, D), lambda i: (i, 0)),
            pl.BlockSpec((D, D), lambda i: (0, 0)),
        ],
        out_specs=pl.BlockSpec((block_rows, D), lambda i: (i, 0)),
        out_shape=jax.ShapeDtypeStruct((v, D), jnp.float32),
    )(table, w_a)


_NC = 2
_NS = 16
_NW = _NC * _NS
_CHUNK = 128
_TPW = T // _NW
_NCHUNK = _TPW // _CHUNK


def _gather_add_sc(starts_f, paths_f, ends_f, tvs, tp, tve):
    mesh = plsc.VectorSubcoreMesh(core_axis_name="c", subcore_axis_name="s")

    @functools.partial(
        pl.kernel,
        mesh=mesh,
        out_type=jax.ShapeDtypeStruct((T, D), jnp.float32),
        scratch_types=[
            pltpu.VMEM((_CHUNK,), jnp.int32),
            pltpu.VMEM((_CHUNK,), jnp.int32),
            pltpu.VMEM((_CHUNK,), jnp.int32),
            pltpu.VMEM((_CHUNK, D), jnp.float32),
            pltpu.VMEM((_CHUNK, D), jnp.float32),
            pltpu.VMEM((_CHUNK, D), jnp.float32),
            pltpu.SemaphoreType.DMA,
            pltpu.SemaphoreType.DMA,
            pltpu.SemaphoreType.DMA,
        ],
    )
    def k(starts_hbm, paths_hbm, ends_hbm, tvs_hbm, tp_hbm, tve_hbm, out_hbm,
          idx_s, idx_p, idx_e, rs, rp, re_, sem1, sem2, sem3):
        wid = lax.axis_index("s") * _NC + lax.axis_index("c")
        wbase = wid * _TPW

        def chunk_body(ci, carry):
            base = wbase + ci * _CHUNK
            pltpu.sync_copy(starts_hbm.at[pl.ds(base, _CHUNK)], idx_s)
            pltpu.sync_copy(paths_hbm.at[pl.ds(base, _CHUNK)], idx_p)
            pltpu.sync_copy(ends_hbm.at[pl.ds(base, _CHUNK)], idx_e)
            c1 = pltpu.async_copy(tvs_hbm.at[idx_s], rs, sem1)
            c2 = pltpu.async_copy(tp_hbm.at[idx_p], rp, sem2)
            c3 = pltpu.async_copy(tve_hbm.at[idx_e], re_, sem3)
            c1.wait()
            c2.wait()
            c3.wait()

            def tok_body(t, acc):
                for j in range(D // 16):
                    sl = pl.ds(j * 16, 16)
                    rs[t, sl] = rs[t, sl] + rp[t, sl] + re_[t, sl]
                return acc

            lax.fori_loop(0, _CHUNK, tok_body, 0)
            pltpu.sync_copy(rs, out_hbm.at[pl.ds(base, _CHUNK)])
            return carry

        lax.fori_loop(0, _NCHUNK, chunk_body, 0)

    return k(starts_f, paths_f, ends_f, tvs, tp, tve)


_BB = 8


def _attn_body(pre_ref, starts_ref, a_ref, wout_ref, code_ref, out_ref):
    h = jnp.tanh(pre_ref[...])
    mask = (starts_ref[...] > 1).astype(jnp.float32)
    scores = jnp.sum(h * a_ref[...][None, :, :], axis=2)
    s = scores * mask + (1.0 - mask) * NEG_INF
    mx = jnp.max(s, axis=1, keepdims=True)
    e = jnp.exp(s - mx)
    w = e / jnp.sum(e, axis=1, keepdims=True)
    code = jnp.sum(h * w[:, :, None], axis=1)
    code_ref[...] = code
    out_ref[...] = jnp.dot(code, wout_ref[...], preferred_element_type=jnp.float32)


def _attention(pre, starts, a, w_out):
    grid = B // _BB
    return pl.pallas_call(
        _attn_body,
        grid=(grid,),
        in_specs=[
            pl.BlockSpec((_BB, NP_, D), lambda i: (i, 0, 0)),
            pl.BlockSpec((_BB, NP_), lambda i: (i, 0)),
            pl.BlockSpec((1, D), lambda i: (0, 0)),
            pl.BlockSpec((D, LABELS), lambda i: (0, 0)),
        ],
        out_specs=[
            pl.BlockSpec((_BB, D), lambda i: (i, 0)),
            pl.BlockSpec((_BB, LABELS), lambda i: (i, 0)),
        ],
        out_shape=[
            jax.ShapeDtypeStruct((B, D), jnp.float32),
            jax.ShapeDtypeStruct((B, LABELS), jnp.float32),
        ],
    )(pre, starts, a, w_out)


def kernel(starts, paths, ends, values_table, paths_table, W, a, W_out):
    w_s = W[0:D]
    w_p = W[D:2 * D]
    w_e = W[2 * D:3 * D]
    tvs, tve = _transform_values(values_table, w_s, w_e)
    tp = _transform_paths(paths_table, w_p)
    pre_flat = _gather_add_sc(
        starts.reshape(-1), paths.reshape(-1), ends.reshape(-1), tvs, tp, tve)
    pre = pre_flat.reshape(B, NP_, D)
    code, out = _attention(pre, starts, a, W_out)
    return (code, out)


# speedup vs baseline: 5.0813x; 1.4618x over previous
import functools

import jax
import jax.numpy as jnp
from jax import lax
from jax.experimental import pallas as pl
from jax.experimental.pallas import tpu as pltpu
from jax.experimental.pallas import tpu_sc as plsc

B = 1024
NP_ = 200
D = 128
LABELS = 1000
T = B * NP_
NEG_INF = -2.0 * 10**10


def _transform2_body(tbl_ref, wa_ref, wb_ref, oa_ref, ob_ref):
    blk = tbl_ref[...]
    oa_ref[...] = jnp.dot(blk, wa_ref[...], preferred_element_type=jnp.float32)
    ob_ref[...] = jnp.dot(blk, wb_ref[...], preferred_element_type=jnp.float32)


def _transform1_body(tbl_ref, wa_ref, oa_ref):
    oa_ref[...] = jnp.dot(tbl_ref[...], wa_ref[...], preferred_element_type=jnp.float32)


def _transform_values(table, w_a, w_b, block_rows=2000):
    v = table.shape[0]
    grid = v // block_rows
    return pl.pallas_call(
        _transform2_body,
        grid=(grid,),
        in_specs=[
            pl.BlockSpec((block_rows, D), lambda i: (i, 0)),
            pl.BlockSpec((D, D), lambda i: (0, 0)),
            pl.BlockSpec((D, D), lambda i: (0, 0)),
        ],
        out_specs=[
            pl.BlockSpec((block_rows, D), lambda i: (i, 0)),
            pl.BlockSpec((block_rows, D), lambda i: (i, 0)),
        ],
        out_shape=[
            jax.ShapeDtypeStruct((v, D), jnp.float32),
            jax.ShapeDtypeStruct((v, D), jnp.float32),
        ],
    )(table, w_a, w_b)


def _transform_paths(table, w_a, block_rows=2000):
    v = table.shape[0]
    grid = v // block_rows
    return pl.pallas_call(
        _transform1_body,
        grid=(grid,),
        in_specs=[
            pl.BlockSpec((block_rows, D), lambda i: (i, 0)),
            pl.BlockSpec((D, D), lambda i: (0, 0)),
        ],
        out_specs=pl.BlockSpec((block_rows, D), lambda i: (i, 0)),
        out_shape=jax.ShapeDtypeStruct((v, D), jnp.float32),
    )(table, w_a)


_NC = 2
_NS = 16
_NW = _NC * _NS
_CHUNK = 128
_TPW = T // _NW
_NCHUNK = _TPW // _CHUNK
_NPAIR = _NCHUNK // 2


def _gather_add_sc(idx_all, tvs, tp, tve):
    mesh = plsc.VectorSubcoreMesh(core_axis_name="c", subcore_axis_name="s")

    @functools.partial(
        pl.kernel,
        mesh=mesh,
        out_type=jax.ShapeDtypeStruct((T, D), jnp.float32),
        scratch_types=[
            pltpu.VMEM((3, _CHUNK), jnp.int32),
            pltpu.VMEM((3, _CHUNK), jnp.int32),
            pltpu.VMEM((_CHUNK, D), jnp.float32),
            pltpu.VMEM((_CHUNK, D), jnp.float32),
            pltpu.VMEM((_CHUNK, D), jnp.float32),
            pltpu.VMEM((_CHUNK, D), jnp.float32),
            pltpu.VMEM((_CHUNK, D), jnp.float32),
            pltpu.VMEM((_CHUNK, D), jnp.float32),
            pltpu.SemaphoreType.DMA,
            pltpu.SemaphoreType.DMA,
            pltpu.SemaphoreType.DMA,
            pltpu.SemaphoreType.DMA,
            pltpu.SemaphoreType.DMA,
            pltpu.SemaphoreType.DMA,
        ],
    )
    def k(idx_hbm, tvs_hbm, tp_hbm, tve_hbm, out_hbm,
          idx_a, idx_b, rs_a, rp_a, re_a, rs_b, rp_b, re_b,
          s1a, s2a, s3a, s1b, s2b, s3b):
        wid = lax.axis_index("s") * _NC + lax.axis_index("c")
        wchunk0 = wid * _NCHUNK

        def fetch(ci, idx_v, rs, rp, re_, s1, s2, s3):
            pltpu.sync_copy(idx_hbm.at[wchunk0 + ci], idx_v)
            c1 = pltpu.async_copy(tvs_hbm.at[idx_v.at[0]], rs, s1)
            c2 = pltpu.async_copy(tp_hbm.at[idx_v.at[1]], rp, s2)
            c3 = pltpu.async_copy(tve_hbm.at[idx_v.at[2]], re_, s3)
            return c1, c2, c3

        def drain(ci, cs, rs, rp, re_):
            for c in cs:
                c.wait()

            def tok_body(t, acc):
                for j in range(D // 16):
                    sl = pl.ds(j * 16, 16)
                    rs[t, sl] = rs[t, sl] + rp[t, sl] + re_[t, sl]
                return acc

            lax.fori_loop(0, _CHUNK, tok_body, 0)
            pltpu.sync_copy(rs, out_hbm.at[pl.ds((wchunk0 + ci) * _CHUNK, _CHUNK)])

        fetch(0, idx_a, rs_a, rp_a, re_a, s1a, s2a, s3a)

        def pair_body(g, carry):
            ca = 2 * g
            cb = 2 * g + 1
            cs_b = fetch(cb, idx_b, rs_b, rp_b, re_b, s1b, s2b, s3b)
            cs_a = (pltpu.make_async_copy(tvs_hbm.at[idx_a.at[0]], rs_a, s1a),
                    pltpu.make_async_copy(tp_hbm.at[idx_a.at[1]], rp_a, s2a),
                    pltpu.make_async_copy(tve_hbm.at[idx_a.at[2]], re_a, s3a))
            drain(ca, cs_a, rs_a, rp_a, re_a)

            @pl.when(g < _NPAIR - 1)
            def _():
                fetch(ca + 2, idx_a, rs_a, rp_a, re_a, s1a, s2a, s3a)

            drain(cb, cs_b, rs_b, rp_b, re_b)
            return carry

        lax.fori_loop(0, _NPAIR, pair_body, 0)

    return k(idx_all, tvs, tp, tve)


_BB = 16


def _attn_body(pre_ref, starts_ref, at_ref, wout_ref, code_ref, out_ref):
    h2 = jnp.tanh(pre_ref[...].reshape(_BB * NP_, D))
    scores = jnp.dot(h2, at_ref[...], preferred_element_type=jnp.float32)
    s = scores.reshape(_BB, NP_)
    mask = (starts_ref[...] > 1).astype(jnp.float32)
    s = s * mask + (1.0 - mask) * NEG_INF
    mx = jnp.max(s, axis=1, keepdims=True)
    e = jnp.exp(s - mx)
    w = e / jnp.sum(e, axis=1, keepdims=True)
    row = lax.broadcasted_iota(jnp.int32, (_BB, _BB * NP_), 0)
    col = lax.broadcasted_iota(jnp.int32, (_BB, _BB * NP_), 1) // NP_
    wd = jnp.where(row == col,
                   jnp.broadcast_to(w.reshape(1, _BB * NP_), (_BB, _BB * NP_)),
                   0.0)
    code = jnp.dot(wd, h2, preferred_element_type=jnp.float32)
    code_ref[...] = code
    out_ref[...] = jnp.dot(code, wout_ref[...], preferred_element_type=jnp.float32)


def _attention(pre, starts, a_t, w_out):
    grid = B // _BB
    return pl.pallas_call(
        _attn_body,
        grid=(grid,),
        in_specs=[
            pl.BlockSpec((_BB, NP_, D), lambda i: (i, 0, 0)),
            pl.BlockSpec((_BB, NP_), lambda i: (i, 0)),
            pl.BlockSpec((D, 1), lambda i: (0, 0)),
            pl.BlockSpec((D, LABELS), lambda i: (0, 0)),
        ],
        out_specs=[
            pl.BlockSpec((_BB, D), lambda i: (i, 0)),
            pl.BlockSpec((_BB, LABELS), lambda i: (i, 0)),
        ],
        out_shape=[
            jax.ShapeDtypeStruct((B, D), jnp.float32),
            jax.ShapeDtypeStruct((B, LABELS), jnp.float32),
        ],
    )(pre, starts, a_t, w_out)


def kernel(starts, paths, ends, values_table, paths_table, W, a, W_out):
    w_s = W[0:D]
    w_p = W[D:2 * D]
    w_e = W[2 * D:3 * D]
    tvs, tve = _transform_values(values_table, w_s, w_e)
    tp = _transform_paths(paths_table, w_p)
    idx_all = jnp.stack(
        [starts.reshape(-1, _CHUNK),
         paths.reshape(-1, _CHUNK),
         ends.reshape(-1, _CHUNK)], axis=1)
    pre_flat = _gather_add_sc(idx_all, tvs, tp, tve)
    pre = pre_flat.reshape(B, NP_, D)
    code, out = _attention(pre, starts, a.T, W_out)
    return (code, out)
